# baseline (speedup 1.0000x reference)
import functools
import math

import jax
import jax.numpy as jnp
from jax import lax
from jax.experimental import pallas as pl
from jax.experimental.pallas import tpu as pltpu
from jax.experimental.pallas import tpu_sc as plsc

S = 64.0
M = 0.6
_LN2 = 0.6931471805599453

_GR = 16
_NBUF = 3
_LANES = 16


def _sc_ln(y):
    bits = lax.bitcast_convert_type(y, jnp.int32)
    e = ((bits >> 23) & 0xFF) - 127
    m = lax.bitcast_convert_type((bits & 0x7FFFFF) | 0x3F800000, jnp.float32)
    z = (m - 1.0) / (m + 1.0)
    z2 = z * z
    p = jnp.float32(1.0 / 13.0)
    for c in (1.0 / 11.0, 1.0 / 9.0, 1.0 / 7.0, 1.0 / 5.0, 1.0 / 3.0, 1.0):
        p = p * z2 + jnp.float32(c)
    return e.astype(jnp.float32) * _LN2 + 2.0 * z * p


def _acos01_sc(x):
    p = jnp.float32(-0.0012624911)
    for c in (0.0066700901, -0.0170881256, 0.0308918810, -0.0501743046,
              0.0889789874, -0.2145988016, 1.5707963050):
        p = p * x + jnp.float32(c)
    a = 1.0 - x
    root = jnp.exp(0.5 * _sc_ln(a))
    return root * p


def _cos_0_pi(z):
    u = 0.5 * z
    u2 = u * u
    s = jnp.float32(1.0 / 362880.0)
    for c in (-1.0 / 5040.0, 1.0 / 120.0, -1.0 / 6.0, 1.0):
        s = s * u2 + jnp.float32(c)
    s = s * u
    return 1.0 - 2.0 * s * s


def _powerface_scaled(vec_scaled):
    x = vec_scaled * jnp.float32(1.0 / S)
    t = _acos01_sc(x)
    y = jnp.exp(M * _sc_ln(t * jnp.float32(1.0 / math.pi)))
    return _cos_0_pi(y * math.pi) * S


def _make_sc_kernel(VR, C):
    info = plsc.get_sparse_core_info()
    NC = info.num_cores
    NW = NC * info.num_subcores
    NGRP = VR // _GR
    FULL = NGRP // NW
    TAILN = NGRP - FULL * NW
    mesh = plsc.VectorSubcoreMesh(core_axis_name="c", subcore_axis_name="s")

    @functools.partial(
        pl.kernel,
        mesh=mesh,
        out_type=jax.ShapeDtypeStruct((VR, C), jnp.float32),
        scratch_types=[
            pltpu.VMEM((_NBUF, _GR, C), jnp.float32),
            pltpu.VMEM((_NBUF, _GR, C), jnp.float32),
            pltpu.VMEM((C + _LANES,), jnp.int32),
            pltpu.VMEM((224,), jnp.int32),
            pltpu.VMEM((224,), jnp.int32),
            pltpu.VMEM((C + _LANES,), jnp.int32),
            pltpu.VMEM((C + _LANES,), jnp.int32),
            pltpu.SemaphoreType.DMA((_NBUF,)),
            pltpu.SemaphoreType.DMA((_NBUF,)),
        ],
    )
    def sc_k(x_hbm, lbl_hbm, o_hbm, inb, outb, lblv, jcnt, joff, orow, ocol,
             isem, osem):
        wid = lax.axis_index("s") * NC + lax.axis_index("c")
        lane_iota = lax.iota(jnp.int32, _LANES)

        pltpu.sync_copy(lbl_hbm, lblv.at[pl.ds(0, C)])

        lane0 = lane_iota == 0

        def lane0_write(ref, pos, val):
            vec = ref[pl.ds(pos, _LANES)]
            ref[pl.ds(pos, _LANES)] = jnp.where(
                lane0, jnp.broadcast_to(val, (_LANES,)), vec)

        for i in range(224 // _LANES):
            jcnt[pl.ds(i * _LANES, _LANES)] = jnp.broadcast_to(
                jnp.int32(0), (_LANES,))

        def count(r, carry):
            R = lblv[pl.ds(r, _LANES)][0]
            gid = R // _GR
            own = (gid % NW) == wid

            @pl.when(own)
            def _():
                j = (gid - wid) // NW
                lane0_write(jcnt, j, jcnt[pl.ds(j, _LANES)][0] + 1)
            return carry

        lax.fori_loop(0, C, count, 0)

        def prefix(j, acc):
            lane0_write(joff, j, acc)
            return acc + jcnt[pl.ds(j, _LANES)][0]

        lax.fori_loop(0, FULL + 1, prefix, 0)

        def place(r, carry):
            R = lblv[pl.ds(r, _LANES)][0]
            gid = R // _GR
            own = (gid % NW) == wid

            @pl.when(own)
            def _():
                j = (gid - wid) // NW
                pos = joff[pl.ds(j, _LANES)][0]
                lane0_write(orow, pos, R)
                lane0_write(ocol, pos, r)
                lane0_write(joff, j, pos + 1)
            return carry

        lax.fori_loop(0, C, place, 0)

        def patch_block(j, s):
            outbs = outb.at[s]
            base = (j * NW + wid) * _GR
            end = joff[pl.ds(j, _LANES)][0]
            start = end - jcnt[pl.ds(j, _LANES)][0]

            def body(k, carry):
                R = orow[pl.ds(k, _LANES)][0]
                r = ocol[pl.ds(k, _LANES)][0]
                rloc = R - base
                gb = (r // _LANES) * _LANES
                lane = r % _LANES
                for rr in range(_GR):
                    @pl.when(rloc == rr)
                    def _(rr=rr):
                        vec = outbs[rr, pl.ds(gb, _LANES)]
                        newv = _powerface_scaled(vec)
                        outbs[rr, pl.ds(gb, _LANES)] = jnp.where(
                            lane_iota == lane, newv, vec)
                return carry

            lax.fori_loop(start, end, body, 0)

        def grow(j):
            return (j * NW + wid) * _GR

        def start_in(j, s):
            pltpu.make_async_copy(
                x_hbm.at[pl.ds(grow(j), _GR)], inb.at[s], isem.at[s]).start()

        def wait_in(j, s):
            pltpu.make_async_copy(
                x_hbm.at[pl.ds(grow(j), _GR)], inb.at[s], isem.at[s]).wait()

        def start_out(j, s):
            pltpu.make_async_copy(
                outb.at[s], o_hbm.at[pl.ds(grow(j), _GR)], osem.at[s]).start()

        def wait_out(j, s):
            pltpu.make_async_copy(
                outb.at[s], o_hbm.at[pl.ds(grow(j), _GR)], osem.at[s]).wait()

        def scale_block(src, dst):
            for rr in range(_GR):
                @plsc.parallel_loop(0, C // _LANES, unroll=8)
                def _body(i):
                    dst[rr, pl.ds(i * _LANES, _LANES)] = (
                        src[rr, pl.ds(i * _LANES, _LANES)] * S)

        for g in range(_NBUF):
            start_in(g, g)

        def step(j, carry):
            s = j % _NBUF
            wait_in(j, s)

            @pl.when(j >= _NBUF)
            def _():
                wait_out(j - _NBUF, s)

            scale_block(inb.at[s], outb.at[s])
            patch_block(j, s)
            start_out(j, s)

            @pl.when(j + _NBUF < FULL)
            def _():
                start_in(j + _NBUF, s)
            return carry

        lax.fori_loop(0, FULL, step, 0)
        for j in range(FULL - _NBUF, FULL):
            wait_out(j, j % _NBUF)

        @pl.when(wid < TAILN)
        def _():
            r = (FULL * NW + wid) * _GR
            pltpu.make_async_copy(
                x_hbm.at[pl.ds(r, _GR)], inb.at[0], isem.at[0]).start()
            pltpu.make_async_copy(
                x_hbm.at[pl.ds(r, _GR)], inb.at[0], isem.at[0]).wait()
            scale_block(inb.at[0], outb.at[0])
            patch_block(FULL, 0)
            pltpu.make_async_copy(
                outb.at[0], o_hbm.at[pl.ds(r, _GR)], osem.at[0]).start()
            pltpu.make_async_copy(
                outb.at[0], o_hbm.at[pl.ds(r, _GR)], osem.at[0]).wait()

    return sc_k


def kernel(logits, labels):
    B, V = logits.shape
    lbl = labels.astype(jnp.int32)
    out_t = _make_sc_kernel(V, B)(logits.T, lbl)
    return out_t.T


# speedup vs baseline: 1.0909x; 1.0909x over previous
import functools
import math

import jax
import jax.numpy as jnp
from jax import lax
from jax.experimental import pallas as pl
from jax.experimental.pallas import tpu as pltpu
from jax.experimental.pallas import tpu_sc as plsc

S = 64.0
M = 0.6
_LN2 = 0.6931471805599453

_GR = 16
_NBUF = 3
_LANES = 16


def _sc_ln(y):
    bits = lax.bitcast_convert_type(y, jnp.int32)
    e = ((bits >> 23) & 0xFF) - 127
    m = lax.bitcast_convert_type((bits & 0x7FFFFF) | 0x3F800000, jnp.float32)
    z = (m - 1.0) / (m + 1.0)
    z2 = z * z
    p = jnp.float32(1.0 / 13.0)
    for c in (1.0 / 11.0, 1.0 / 9.0, 1.0 / 7.0, 1.0 / 5.0, 1.0 / 3.0, 1.0):
        p = p * z2 + jnp.float32(c)
    return e.astype(jnp.float32) * _LN2 + 2.0 * z * p


def _acos01_sc(x):
    p = jnp.float32(-0.0012624911)
    for c in (0.0066700901, -0.0170881256, 0.0308918810, -0.0501743046,
              0.0889789874, -0.2145988016, 1.5707963050):
        p = p * x + jnp.float32(c)
    a = 1.0 - x
    root = jnp.exp(0.5 * _sc_ln(a))
    return root * p


def _cos_0_pi(z):
    u = 0.5 * z
    u2 = u * u
    s = jnp.float32(1.0 / 362880.0)
    for c in (-1.0 / 5040.0, 1.0 / 120.0, -1.0 / 6.0, 1.0):
        s = s * u2 + jnp.float32(c)
    s = s * u
    return 1.0 - 2.0 * s * s


def _powerface_scaled(vec_scaled):
    x = vec_scaled * jnp.float32(1.0 / S)
    t = _acos01_sc(x)
    y = jnp.exp(M * _sc_ln(t * jnp.float32(1.0 / math.pi)))
    return _cos_0_pi(y * math.pi) * S


def _make_sc_kernel(VR, C):
    info = plsc.get_sparse_core_info()
    NC = info.num_cores
    NW = NC * info.num_subcores
    NGRP = VR // _GR
    FULL = NGRP // NW
    TAILN = NGRP - FULL * NW
    mesh = plsc.VectorSubcoreMesh(core_axis_name="c", subcore_axis_name="s")

    @functools.partial(
        pl.kernel,
        mesh=mesh,
        out_type=jax.ShapeDtypeStruct((VR, C), jnp.float32),
        scratch_types=[
            pltpu.VMEM((_NBUF, _GR, C), jnp.float32),
            pltpu.VMEM((_NBUF, _GR, C), jnp.float32),
            pltpu.VMEM((C + _LANES,), jnp.int32),
            pltpu.VMEM((224,), jnp.int32),
            pltpu.VMEM((224,), jnp.int32),
            pltpu.VMEM((C + _LANES,), jnp.int32),
            pltpu.VMEM((C + _LANES,), jnp.int32),
            pltpu.VMEM((C + _LANES,), jnp.int32),
            pltpu.SemaphoreType.DMA((_NBUF,)),
            pltpu.SemaphoreType.DMA((_NBUF,)),
        ],
    )
    def sc_k(x_hbm, lbl_hbm, o_hbm, inb, outb, lblv, jcnt, headv, nxt, orow, ocol,
             isem, osem):
        wid = lax.axis_index("s") * NC + lax.axis_index("c")
        lane_iota = lax.iota(jnp.int32, _LANES)

        pltpu.sync_copy(lbl_hbm, lblv.at[pl.ds(0, C)])

        lane0 = lane_iota == 0

        def lane0_write(ref, pos, val):
            vec = ref[pl.ds(pos, _LANES)]
            ref[pl.ds(pos, _LANES)] = jnp.where(
                lane0, jnp.broadcast_to(val, (_LANES,)), vec)

        for i in range(224 // _LANES):
            jcnt[pl.ds(i * _LANES, _LANES)] = jnp.broadcast_to(
                jnp.int32(0), (_LANES,))

        def build(r, cursor):
            R = lblv[pl.ds(r, _LANES)][0]
            gid = R // _GR
            own = (gid % NW) == wid

            @pl.when(own)
            def _():
                j = (gid - wid) // NW
                lane0_write(jcnt, j, jcnt[pl.ds(j, _LANES)][0] + 1)
                lane0_write(nxt, cursor, headv[pl.ds(j, _LANES)][0])
                lane0_write(headv, j, cursor)
                lane0_write(orow, cursor, R)
                lane0_write(ocol, cursor, r)
            return cursor + jnp.where(own, 1, 0)

        def run_build():
            for i in range(224 // _LANES):
                headv[pl.ds(i * _LANES, _LANES)] = jnp.broadcast_to(
                    jnp.int32(-1), (_LANES,))
            lax.fori_loop(0, C, build, 0)

        def patch_block(j, s):
            outbs = outb.at[s]
            base = (j * NW + wid) * _GR
            cnt = jcnt[pl.ds(j, _LANES)][0]
            k0 = headv[pl.ds(j, _LANES)][0]

            def body(i, k):
                R = orow[pl.ds(k, _LANES)][0]
                r = ocol[pl.ds(k, _LANES)][0]
                rloc = R - base
                gb = (r // _LANES) * _LANES
                lane = r % _LANES
                for rr in range(_GR):
                    @pl.when(rloc == rr)
                    def _(rr=rr):
                        vec = outbs[rr, pl.ds(gb, _LANES)]
                        newv = _powerface_scaled(vec)
                        outbs[rr, pl.ds(gb, _LANES)] = jnp.where(
                            lane_iota == lane, newv, vec)
                return nxt[pl.ds(k, _LANES)][0]

            lax.fori_loop(0, cnt, body, k0)

        def grow(j):
            return (j * NW + wid) * _GR

        def start_in(j, s):
            pltpu.make_async_copy(
                x_hbm.at[pl.ds(grow(j), _GR)], inb.at[s], isem.at[s]).start()

        def wait_in(j, s):
            pltpu.make_async_copy(
                x_hbm.at[pl.ds(grow(j), _GR)], inb.at[s], isem.at[s]).wait()

        def start_out(j, s):
            pltpu.make_async_copy(
                outb.at[s], o_hbm.at[pl.ds(grow(j), _GR)], osem.at[s]).start()

        def wait_out(j, s):
            pltpu.make_async_copy(
                outb.at[s], o_hbm.at[pl.ds(grow(j), _GR)], osem.at[s]).wait()

        def scale_block(src, dst):
            for rr in range(_GR):
                @plsc.parallel_loop(0, C // _LANES, unroll=8)
                def _body(i):
                    dst[rr, pl.ds(i * _LANES, _LANES)] = (
                        src[rr, pl.ds(i * _LANES, _LANES)] * S)

        for g in range(_NBUF):
            start_in(g, g)
        run_build()

        def step(j, carry):
            s = j % _NBUF
            wait_in(j, s)

            @pl.when(j >= _NBUF)
            def _():
                wait_out(j - _NBUF, s)

            scale_block(inb.at[s], outb.at[s])
            patch_block(j, s)
            start_out(j, s)

            @pl.when(j + _NBUF < FULL)
            def _():
                start_in(j + _NBUF, s)
            return carry

        lax.fori_loop(0, FULL, step, 0)
        for j in range(FULL - _NBUF, FULL):
            wait_out(j, j % _NBUF)

        @pl.when(wid < TAILN)
        def _():
            r = (FULL * NW + wid) * _GR
            pltpu.make_async_copy(
                x_hbm.at[pl.ds(r, _GR)], inb.at[0], isem.at[0]).start()
            pltpu.make_async_copy(
                x_hbm.at[pl.ds(r, _GR)], inb.at[0], isem.at[0]).wait()
            scale_block(inb.at[0], outb.at[0])
            patch_block(FULL, 0)
            pltpu.make_async_copy(
                outb.at[0], o_hbm.at[pl.ds(r, _GR)], osem.at[0]).start()
            pltpu.make_async_copy(
                outb.at[0], o_hbm.at[pl.ds(r, _GR)], osem.at[0]).wait()

    return sc_k


def kernel(logits, labels):
    B, V = logits.shape
    lbl = labels.astype(jnp.int32)
    out_t = _make_sc_kernel(V, B)(logits.T, lbl)
    return out_t.T
